# baseline (speedup 1.0000x reference)
import jax
import jax.numpy as jnp
from jax import lax
from jax.experimental import pallas as pl
from jax.experimental.pallas import tpu as pltpu
from jax.experimental.pallas import tpu_sc as plsc

_B, _T = 1024, 200
_N = _B * _T
_EMB = 64
_OUT_D = 3 * _EMB + 32
_EPS = 1e-5

_T0, _T1 = 96, 104
_N0, _N1 = _T0 * _B, _T1 * _B

_NC, _NS = 2, 16
_NW = _NC * _NS
_IDXROW = 128


def _make_sc_body(start, nrows, ch_rows):
    rows_per_w = nrows // _NW
    ch = ch_rows * _IDXROW
    nch = rows_per_w // ch

    def body(i0_hbm, i1_hbm, i2_hbm, t0_hbm, t1_hbm, t2_hbm,
             o01_hbm, o2_hbm, idx_v, rows_v, sem):
        wid = lax.axis_index("s") * _NC + lax.axis_index("c")
        gbase = start + wid * rows_per_w
        obase = wid * rows_per_w

        for ih, th, oh, col in ((i0_hbm, t0_hbm, o01_hbm, 0),
                                (i1_hbm, t1_hbm, o01_hbm, _EMB),
                                (i2_hbm, t2_hbm, o2_hbm, 0)):
            @pl.loop(0, nch)
            def _(c):
                r0 = gbase + c * ch
                q0 = obase + c * ch
                pltpu.sync_copy(ih.at[pl.ds(r0, ch)], idx_v)
                cps = []
                for j in range(ch_rows):
                    cps.append(pltpu.async_copy(
                        th.at[idx_v.at[pl.ds(j * _IDXROW, _IDXROW)]],
                        rows_v.at[pl.ds(j * _IDXROW, _IDXROW)], sem))
                for cp in cps:
                    cp.wait()
                pltpu.sync_copy(rows_v,
                                oh.at[pl.ds(q0, ch), pl.ds(col, _EMB)])
    return body


def _sc_gather(start, nrows, ch_rows, idx0, idx1, idx2, tab0, tab1, tab2):
    mesh = plsc.VectorSubcoreMesh(core_axis_name="c", subcore_axis_name="s",
                                  num_cores=_NC, num_subcores=_NS)
    ch = ch_rows * _IDXROW
    wide_ty = jax.ShapeDtypeStruct((nrows, 2 * _EMB), jnp.float32)
    k = pl.kernel(
        _make_sc_body(start, nrows, ch_rows),
        out_type=(wide_ty, wide_ty),
        mesh=mesh,
        scratch_types=[
            pltpu.VMEM((ch,), jnp.int32),
            pltpu.VMEM((ch, _EMB), jnp.float32),
            pltpu.SemaphoreType.DMA,
        ],
        compiler_params=pltpu.CompilerParams(use_tc_tiling_on_sc=False),
    )
    return k(idx0, idx1, idx2, tab0, tab1, tab2)


_TB = 8


def _make_fuse_body(t_base, aliased):
    def body(*refs):
        if aliased:
            _, c01_ref, c2_ref, nf0_ref, nf1_ref, seq_ref, wb_ref, \
                scal_ref, out_ref = refs
        else:
            c01_ref, c2_ref, nf0_ref, nf1_ref, seq_ref, wb_ref, \
                scal_ref, out_ref = refs
        i = pl.program_id(0)
        seq = seq_ref[...]
        iota_t = lax.broadcasted_iota(jnp.int32, (_T, _B), 0)
        mfull = (iota_t < seq).astype(jnp.float32)
        cnt = jnp.maximum(jnp.sum(mfull), 1.0)

        t0 = i * _TB + t_base
        iota_b = lax.broadcasted_iota(jnp.int32, (_TB, _B), 0) + t0
        mask = iota_b < seq

        c01 = jnp.swapaxes(c01_ref[...], 1, 2)
        c2 = jnp.swapaxes(c2_ref[...], 1, 2)[:, :_EMB, :]
        pieces = [c01, c2]
        for f, ref in enumerate((nf0_ref, nf1_ref)):
            xf = ref[...]
            s1 = jnp.sum(xf * mfull)
            s2 = jnp.sum(xf * xf * mfull)
            mean = s1 / cnt
            var = jnp.maximum(s2 / cnt - mean * mean, 0.0)
            rstd = lax.rsqrt(var + _EPS)
            gamma = scal_ref[2 * f]
            beta = scal_ref[2 * f + 1]
            x = ref[pl.ds(t0, _TB), :]
            xn = (x - mean) * (rstd * gamma) + beta
            y = jnp.where(mask, xn, x)
            w = wb_ref[2 * f, :]
            b = wb_ref[2 * f + 1, :]
            pieces.append(y[:, None, :] * w[None, :, None]
                          + b[None, :, None])
        out_ref[...] = jnp.concatenate(pieces, axis=1)
    return body


def _tc_fuse(t_base, nsteps, c01, c2, nf0, nf1, seq2d, wb, scal, prev=None):
    blk_base = t_base // _TB
    cat_spec = pl.BlockSpec((_TB, _B, 2 * _EMB), lambda i: (i, 0, 0))
    in_specs = [
        cat_spec, cat_spec,
        pl.BlockSpec((_T, _B), lambda i: (0, 0)),
        pl.BlockSpec((_T, _B), lambda i: (0, 0)),
        pl.BlockSpec((1, _B), lambda i: (0, 0)),
        pl.BlockSpec((4, 16), lambda i: (0, 0)),
        pl.BlockSpec(memory_space=pltpu.SMEM),
    ]
    args = [c01, c2, nf0, nf1, seq2d, wb, scal]
    aliases = {}
    if prev is not None:
        in_specs = [pl.BlockSpec(memory_space=pl.ANY)] + in_specs
        args = [prev] + args
        aliases = {0: 0}
    return pl.pallas_call(
        _make_fuse_body(t_base, prev is not None),
        grid=(nsteps,),
        in_specs=in_specs,
        out_specs=pl.BlockSpec((_TB, _OUT_D, _B),
                               lambda i: (i + blk_base, 0, 0)),
        out_shape=jax.ShapeDtypeStruct((_T, _OUT_D, _B), jnp.float32),
        input_output_aliases=aliases,
    )(*args)


def kernel(emb_feat_0, emb_feat_1, emb_feat_2, num_feat_0, num_feat_1,
           event_time, seq_lens, emb_table_0, emb_table_1, emb_table_2,
           bn_gamma_0, bn_beta_0, bn_gamma_1, bn_beta_1,
           lin_w_0, lin_b_0, lin_w_1, lin_b_1):
    idx0 = jnp.transpose(emb_feat_0.astype(jnp.int32), (1, 0)).reshape(_N)
    idx1 = jnp.transpose(emb_feat_1.astype(jnp.int32), (1, 0)).reshape(_N)
    idx2 = jnp.transpose(emb_feat_2.astype(jnp.int32), (1, 0)).reshape(_N)

    tabs = (emb_table_0, emb_table_1, emb_table_2)
    c01_a, c2_a = _sc_gather(0, _N0, 3, idx0, idx1, idx2, *tabs)
    c01_b, c2_b = _sc_gather(_N0, _N1, 2, idx0, idx1, idx2, *tabs)

    seq2d = seq_lens.astype(jnp.int32).reshape(1, _B)
    wb = jnp.stack([lin_w_0[0].astype(jnp.float32),
                    lin_b_0.astype(jnp.float32),
                    lin_w_1[0].astype(jnp.float32),
                    lin_b_1.astype(jnp.float32)], axis=0)
    scal = jnp.stack([bn_gamma_0.astype(jnp.float32),
                      bn_beta_0.astype(jnp.float32),
                      bn_gamma_1.astype(jnp.float32),
                      bn_beta_1.astype(jnp.float32)])

    nf0_t = jnp.transpose(num_feat_0.astype(jnp.float32), (1, 0))
    nf1_t = jnp.transpose(num_feat_1.astype(jnp.float32), (1, 0))

    out_1 = _tc_fuse(0, _T0 // _TB, c01_a.reshape(_T0, _B, 2 * _EMB),
                     c2_a.reshape(_T0, _B, 2 * _EMB),
                     nf0_t, nf1_t, seq2d, wb, scal)
    out_t = _tc_fuse(_T0, _T1 // _TB, c01_b.reshape(_T1, _B, 2 * _EMB),
                     c2_b.reshape(_T1, _B, 2 * _EMB),
                     nf0_t, nf1_t, seq2d, wb, scal, prev=out_1)
    out = jnp.transpose(out_t, (2, 0, 1))
    return out, event_time.astype(jnp.float32)


# speedup vs baseline: 1.1078x; 1.1078x over previous
import jax
import jax.numpy as jnp
from jax import lax
from jax.experimental import pallas as pl
from jax.experimental.pallas import tpu as pltpu
from jax.experimental.pallas import tpu_sc as plsc

_B, _T = 1024, 200
_N = _B * _T
_VOCAB = 100000
_EMB = 64
_OUT_D = 3 * _EMB + 32
_EPS = 1e-5

_T0, _T1 = 80, 120
_N0, _N1 = _T0 * _B, _T1 * _B

_NC, _NS = 2, 16
_NW = _NC * _NS
_IDXROW = 128


def _make_sc_body(start, nrows, ch_rows):
    rows_per_w = nrows // _NW
    ch = ch_rows * _IDXROW
    nch = rows_per_w // ch
    assert nch % 2 == 0

    def body(i0_hbm, i1_hbm, i2_hbm, t0_hbm, t1_hbm, t2_hbm,
             o01_hbm, o2_hbm, idx_v, rows_v, gsem, wsem0, wsem1):
        wid = lax.axis_index("s") * _NC + lax.axis_index("c")
        gbase = start + wid * rows_per_w
        obase = wid * rows_per_w

        for ih, th, oh, col in ((i0_hbm, t0_hbm, o01_hbm, 0),
                                (i1_hbm, t1_hbm, o01_hbm, _EMB),
                                (i2_hbm, t2_hbm, o2_hbm, 0)):
            @pl.loop(0, nch // 2)
            def _(k):
                for p, wsem in ((0, wsem0), (1, wsem1)):
                    c = 2 * k + p
                    r0 = gbase + c * ch
                    q0 = obase + c * ch
                    pltpu.sync_copy(ih.at[pl.ds(r0, ch)], idx_v.at[p])

                    @pl.when(k > 0)
                    def _():
                        pltpu.make_async_copy(
                            rows_v.at[p],
                            oh.at[pl.ds(q0, ch), pl.ds(col, _EMB)],
                            wsem).wait()

                    cps = []
                    for j in range(ch_rows):
                        cps.append(pltpu.async_copy(
                            th.at[idx_v.at[p, pl.ds(j * _IDXROW, _IDXROW)]],
                            rows_v.at[p, pl.ds(j * _IDXROW, _IDXROW)], gsem))
                    for cp in cps:
                        cp.wait()
                    pltpu.async_copy(
                        rows_v.at[p],
                        oh.at[pl.ds(q0, ch), pl.ds(col, _EMB)], wsem)

            for p, wsem in ((0, wsem0), (1, wsem1)):
                q0 = obase + (nch - 2 + p) * ch
                pltpu.make_async_copy(
                    rows_v.at[p],
                    oh.at[pl.ds(q0, ch), pl.ds(col, _EMB)], wsem).wait()
    return body


def _sc_gather(start, nrows, ch_rows, idx0, idx1, idx2, tab0, tab1, tab2):
    mesh = plsc.VectorSubcoreMesh(core_axis_name="c", subcore_axis_name="s",
                                  num_cores=_NC, num_subcores=_NS)
    ch = ch_rows * _IDXROW
    wide_ty = jax.ShapeDtypeStruct((nrows, 2 * _EMB), jnp.float32)
    k = pl.kernel(
        _make_sc_body(start, nrows, ch_rows),
        out_type=(wide_ty, wide_ty),
        mesh=mesh,
        scratch_types=[
            pltpu.VMEM((2, ch), jnp.int32),
            pltpu.VMEM((2, ch, _EMB), jnp.float32),
            pltpu.SemaphoreType.DMA,
            pltpu.SemaphoreType.DMA,
            pltpu.SemaphoreType.DMA,
        ],
        compiler_params=pltpu.CompilerParams(use_tc_tiling_on_sc=False),
    )
    return k(idx0, idx1, idx2, tab0, tab1, tab2)


_TB = 8


def _make_fuse_body(t_base, aliased):
    def body(*refs):
        if aliased:
            _, c01_ref, c2_ref, nf0_ref, nf1_ref, seq_ref, wb_ref, \
                scal_ref, out_ref = refs
        else:
            c01_ref, c2_ref, nf0_ref, nf1_ref, seq_ref, wb_ref, \
                scal_ref, out_ref = refs
        i = pl.program_id(0)
        seq = seq_ref[...]
        iota_t = lax.broadcasted_iota(jnp.int32, (_T, _B), 0)
        mfull = (iota_t < seq).astype(jnp.float32)
        cnt = jnp.maximum(jnp.sum(mfull), 1.0)

        t0 = i * _TB + t_base
        iota_b = lax.broadcasted_iota(jnp.int32, (_TB, _B), 0) + t0
        mask = iota_b < seq

        c01 = jnp.swapaxes(c01_ref[...], 1, 2)
        c2 = jnp.swapaxes(c2_ref[...], 1, 2)[:, :_EMB, :]
        pieces = [c01, c2]
        for f, ref in enumerate((nf0_ref, nf1_ref)):
            xf = ref[...]
            s1 = jnp.sum(xf * mfull)
            s2 = jnp.sum(xf * xf * mfull)
            mean = s1 / cnt
            var = jnp.maximum(s2 / cnt - mean * mean, 0.0)
            rstd = lax.rsqrt(var + _EPS)
            gamma = scal_ref[2 * f]
            beta = scal_ref[2 * f + 1]
            x = ref[pl.ds(t0, _TB), :]
            xn = (x - mean) * (rstd * gamma) + beta
            y = jnp.where(mask, xn, x)
            w = wb_ref[2 * f, :]
            b = wb_ref[2 * f + 1, :]
            pieces.append(y[:, None, :] * w[None, :, None]
                          + b[None, :, None])
        out_ref[...] = jnp.concatenate(pieces, axis=1)
    return body


def _tc_fuse(t_base, nsteps, c01, c2, nf0, nf1, seq2d, wb, scal, prev=None):
    blk_base = t_base // _TB
    cat_spec = pl.BlockSpec((_TB, _B, 2 * _EMB), lambda i: (i, 0, 0))
    in_specs = [
        cat_spec, cat_spec,
        pl.BlockSpec((_T, _B), lambda i: (0, 0)),
        pl.BlockSpec((_T, _B), lambda i: (0, 0)),
        pl.BlockSpec((1, _B), lambda i: (0, 0)),
        pl.BlockSpec((4, 16), lambda i: (0, 0)),
        pl.BlockSpec(memory_space=pltpu.SMEM),
    ]
    args = [c01, c2, nf0, nf1, seq2d, wb, scal]
    aliases = {}
    if prev is not None:
        in_specs = [pl.BlockSpec(memory_space=pl.ANY)] + in_specs
        args = [prev] + args
        aliases = {0: 0}
    return pl.pallas_call(
        _make_fuse_body(t_base, prev is not None),
        grid=(nsteps,),
        in_specs=in_specs,
        out_specs=pl.BlockSpec((_TB, _OUT_D, _B),
                               lambda i: (i + blk_base, 0, 0)),
        out_shape=jax.ShapeDtypeStruct((_T, _OUT_D, _B), jnp.float32),
        input_output_aliases=aliases,
    )(*args)


def kernel(emb_feat_0, emb_feat_1, emb_feat_2, num_feat_0, num_feat_1,
           event_time, seq_lens, emb_table_0, emb_table_1, emb_table_2,
           bn_gamma_0, bn_beta_0, bn_gamma_1, bn_beta_1,
           lin_w_0, lin_b_0, lin_w_1, lin_b_1):
    idx0 = jnp.transpose(emb_feat_0.astype(jnp.int32), (1, 0)).reshape(_N)
    idx1 = jnp.transpose(emb_feat_1.astype(jnp.int32), (1, 0)).reshape(_N)
    idx2 = jnp.transpose(emb_feat_2.astype(jnp.int32), (1, 0)).reshape(_N)

    tabs = (emb_table_0, emb_table_1, emb_table_2)
    c01_a, c2_a = _sc_gather(0, _N0, 5, idx0, idx1, idx2, *tabs)
    c01_b, c2_b = _sc_gather(_N0, _N1, 5, idx0, idx1, idx2, *tabs)

    seq2d = seq_lens.astype(jnp.int32).reshape(1, _B)
    wb = jnp.stack([lin_w_0[0].astype(jnp.float32),
                    lin_b_0.astype(jnp.float32),
                    lin_w_1[0].astype(jnp.float32),
                    lin_b_1.astype(jnp.float32)], axis=0)
    scal = jnp.stack([bn_gamma_0.astype(jnp.float32),
                      bn_beta_0.astype(jnp.float32),
                      bn_gamma_1.astype(jnp.float32),
                      bn_beta_1.astype(jnp.float32)])

    nf0_t = jnp.transpose(num_feat_0.astype(jnp.float32), (1, 0))
    nf1_t = jnp.transpose(num_feat_1.astype(jnp.float32), (1, 0))

    out_1 = _tc_fuse(0, _T0 // _TB, c01_a.reshape(_T0, _B, 2 * _EMB),
                     c2_a.reshape(_T0, _B, 2 * _EMB),
                     nf0_t, nf1_t, seq2d, wb, scal)
    out_t = _tc_fuse(_T0, _T1 // _TB, c01_b.reshape(_T1, _B, 2 * _EMB),
                     c2_b.reshape(_T1, _B, 2 * _EMB),
                     nf0_t, nf1_t, seq2d, wb, scal, prev=out_1)
    out = jnp.transpose(out_t, (2, 0, 1))
    return out, event_time.astype(jnp.float32)


# speedup vs baseline: 1.1308x; 1.0208x over previous
import jax
import jax.numpy as jnp
from jax import lax
from jax.experimental import pallas as pl
from jax.experimental.pallas import tpu as pltpu
from jax.experimental.pallas import tpu_sc as plsc

_B, _T = 1024, 200
_N = _B * _T
_VOCAB = 100000
_EMB = 64
_OUT_D = 3 * _EMB + 32
_EPS = 1e-5

_T0, _T1 = 80, 120
_N0, _N1 = _T0 * _B, _T1 * _B

_NC, _NS = 2, 16
_NW = _NC * _NS
_IDXROW = 128


def _make_sc_body(start, nrows, ch_rows, two_tables):
    rows_per_w = nrows // _NW
    ch = ch_rows * _IDXROW
    nch = rows_per_w // ch
    assert nch % 2 == 0

    def body(*refs):
        if two_tables:
            (i0_hbm, i1_hbm, t0_hbm, t1_hbm, o_hbm,
             idx_v, rows_v, gsem, wsem0, wsem1) = refs
            work = ((i0_hbm, t0_hbm, o_hbm, 0),
                    (i1_hbm, t1_hbm, o_hbm, _EMB))
        else:
            (i2_hbm, t2_hbm, o_hbm,
             idx_v, rows_v, gsem, wsem0, wsem1) = refs
            work = ((i2_hbm, t2_hbm, o_hbm, 0),)
        wid = lax.axis_index("s") * _NC + lax.axis_index("c")
        gbase = start + wid * rows_per_w
        obase = wid * rows_per_w

        for ih, th, oh, col in work:
            @pl.loop(0, nch // 2)
            def _(k):
                for p, wsem in ((0, wsem0), (1, wsem1)):
                    c = 2 * k + p
                    r0 = gbase + c * ch
                    q0 = obase + c * ch
                    pltpu.sync_copy(ih.at[pl.ds(r0, ch)], idx_v.at[p])

                    @pl.when(k > 0)
                    def _():
                        pltpu.make_async_copy(
                            rows_v.at[p],
                            oh.at[pl.ds(q0, ch), pl.ds(col, _EMB)],
                            wsem).wait()

                    cps = []
                    for j in range(ch_rows):
                        cps.append(pltpu.async_copy(
                            th.at[idx_v.at[p, pl.ds(j * _IDXROW, _IDXROW)]],
                            rows_v.at[p, pl.ds(j * _IDXROW, _IDXROW)], gsem))
                    for cp in cps:
                        cp.wait()
                    pltpu.async_copy(
                        rows_v.at[p],
                        oh.at[pl.ds(q0, ch), pl.ds(col, _EMB)], wsem)

            for p, wsem in ((0, wsem0), (1, wsem1)):
                q0 = obase + (nch - 2 + p) * ch
                pltpu.make_async_copy(
                    rows_v.at[p],
                    oh.at[pl.ds(q0, ch), pl.ds(col, _EMB)], wsem).wait()
    return body


def _sc_gather(start, nrows, ch_rows, idxs, tabs):
    mesh = plsc.VectorSubcoreMesh(core_axis_name="c", subcore_axis_name="s",
                                  num_cores=_NC, num_subcores=_NS)
    ch = ch_rows * _IDXROW
    wide_ty = jax.ShapeDtypeStruct((nrows, 2 * _EMB), jnp.float32)
    k = pl.kernel(
        _make_sc_body(start, nrows, ch_rows, len(tabs) == 2),
        out_type=wide_ty,
        mesh=mesh,
        scratch_types=[
            pltpu.VMEM((2, ch), jnp.int32),
            pltpu.VMEM((2, ch, _EMB), jnp.float32),
            pltpu.SemaphoreType.DMA,
            pltpu.SemaphoreType.DMA,
            pltpu.SemaphoreType.DMA,
        ],
        compiler_params=pltpu.CompilerParams(use_tc_tiling_on_sc=False),
    )
    return k(*idxs, *tabs)


_TB = 8


def _make_fuse_body(t_base, aliased):
    def body(*refs):
        if aliased:
            _, c01_ref, c2_ref, nf0_ref, nf1_ref, seq_ref, wb_ref, \
                scal_ref, out_ref = refs
        else:
            c01_ref, c2_ref, nf0_ref, nf1_ref, seq_ref, wb_ref, \
                scal_ref, out_ref = refs
        i = pl.program_id(0)
        seq = seq_ref[...]
        iota_t = lax.broadcasted_iota(jnp.int32, (_T, _B), 0)
        mfull = (iota_t < seq).astype(jnp.float32)
        cnt = jnp.maximum(jnp.sum(mfull), 1.0)

        t0 = i * _TB + t_base
        iota_b = lax.broadcasted_iota(jnp.int32, (_TB, _B), 0) + t0
        mask = iota_b < seq

        c01 = jnp.swapaxes(c01_ref[...], 1, 2)
        c2 = jnp.swapaxes(c2_ref[...], 1, 2)[:, :_EMB, :]
        pieces = [c01, c2]
        for f, ref in enumerate((nf0_ref, nf1_ref)):
            xf = ref[...]
            s1 = jnp.sum(xf * mfull)
            s2 = jnp.sum(xf * xf * mfull)
            mean = s1 / cnt
            var = jnp.maximum(s2 / cnt - mean * mean, 0.0)
            rstd = lax.rsqrt(var + _EPS)
            gamma = scal_ref[2 * f]
            beta = scal_ref[2 * f + 1]
            x = ref[pl.ds(t0, _TB), :]
            xn = (x - mean) * (rstd * gamma) + beta
            y = jnp.where(mask, xn, x)
            w = wb_ref[2 * f, :]
            b = wb_ref[2 * f + 1, :]
            pieces.append(y[:, None, :] * w[None, :, None]
                          + b[None, :, None])
        out_ref[...] = jnp.concatenate(pieces, axis=1)
    return body


def _tc_fuse(t_base, nsteps, c01, c2, nf0, nf1, seq2d, wb, scal, prev=None):
    blk_base = t_base // _TB
    cat_spec = pl.BlockSpec((_TB, _B, 2 * _EMB), lambda i: (i, 0, 0))
    in_specs = [
        cat_spec, cat_spec,
        pl.BlockSpec((_T, _B), lambda i: (0, 0)),
        pl.BlockSpec((_T, _B), lambda i: (0, 0)),
        pl.BlockSpec((1, _B), lambda i: (0, 0)),
        pl.BlockSpec((4, 16), lambda i: (0, 0)),
        pl.BlockSpec(memory_space=pltpu.SMEM),
    ]
    args = [c01, c2, nf0, nf1, seq2d, wb, scal]
    aliases = {}
    if prev is not None:
        in_specs = [pl.BlockSpec(memory_space=pl.ANY)] + in_specs
        args = [prev] + args
        aliases = {0: 0}
    return pl.pallas_call(
        _make_fuse_body(t_base, prev is not None),
        grid=(nsteps,),
        in_specs=in_specs,
        out_specs=pl.BlockSpec((_TB, _OUT_D, _B),
                               lambda i: (i + blk_base, 0, 0)),
        out_shape=jax.ShapeDtypeStruct((_T, _OUT_D, _B), jnp.float32),
        input_output_aliases=aliases,
    )(*args)


def kernel(emb_feat_0, emb_feat_1, emb_feat_2, num_feat_0, num_feat_1,
           event_time, seq_lens, emb_table_0, emb_table_1, emb_table_2,
           bn_gamma_0, bn_beta_0, bn_gamma_1, bn_beta_1,
           lin_w_0, lin_b_0, lin_w_1, lin_b_1):
    idx0 = jnp.transpose(emb_feat_0.astype(jnp.int32), (1, 0)).reshape(_N)
    idx1 = jnp.transpose(emb_feat_1.astype(jnp.int32), (1, 0)).reshape(_N)
    idx2 = jnp.transpose(emb_feat_2.astype(jnp.int32), (1, 0)).reshape(_N)

    c01_a = _sc_gather(0, _N0, 5, (idx0, idx1), (emb_table_0, emb_table_1))
    c2_a = _sc_gather(0, _N0, 5, (idx2,), (emb_table_2,))
    c01_b = _sc_gather(_N0, _N1, 5, (idx0, idx1),
                       (emb_table_0, emb_table_1))
    c2_b = _sc_gather(_N0, _N1, 5, (idx2,), (emb_table_2,))

    seq2d = seq_lens.astype(jnp.int32).reshape(1, _B)
    wb = jnp.stack([lin_w_0[0].astype(jnp.float32),
                    lin_b_0.astype(jnp.float32),
                    lin_w_1[0].astype(jnp.float32),
                    lin_b_1.astype(jnp.float32)], axis=0)
    scal = jnp.stack([bn_gamma_0.astype(jnp.float32),
                      bn_beta_0.astype(jnp.float32),
                      bn_gamma_1.astype(jnp.float32),
                      bn_beta_1.astype(jnp.float32)])

    nf0_t = jnp.transpose(num_feat_0.astype(jnp.float32), (1, 0))
    nf1_t = jnp.transpose(num_feat_1.astype(jnp.float32), (1, 0))

    out_1 = _tc_fuse(0, _T0 // _TB, c01_a.reshape(_T0, _B, 2 * _EMB),
                     c2_a.reshape(_T0, _B, 2 * _EMB),
                     nf0_t, nf1_t, seq2d, wb, scal)
    out_t = _tc_fuse(_T0, _T1 // _TB, c01_b.reshape(_T1, _B, 2 * _EMB),
                     c2_b.reshape(_T1, _B, 2 * _EMB),
                     nf0_t, nf1_t, seq2d, wb, scal, prev=out_1)
    out = jnp.transpose(out_t, (2, 0, 1))
    return out, event_time.astype(jnp.float32)


# speedup vs baseline: 1.1583x; 1.0243x over previous
import jax
import jax.numpy as jnp
from jax import lax
from jax.experimental import pallas as pl
from jax.experimental.pallas import tpu as pltpu
from jax.experimental.pallas import tpu_sc as plsc

_B, _T = 1024, 200
_N = _B * _T
_VOCAB = 100000
_EMB = 64
_OUT_D = 3 * _EMB + 32
_EPS = 1e-5

_T0, _T1 = 80, 120
_N0, _N1 = _T0 * _B, _T1 * _B

_NC, _NS = 2, 16
_NW = _NC * _NS
_IDXROW = 128


def _make_sc_body(start, nrows, ch_rows, two_tables):
    rows_per_w = nrows // _NW
    ch = ch_rows * _IDXROW
    nch = rows_per_w // ch
    assert nch % 2 == 0

    def body(*refs):
        if two_tables:
            (i0_hbm, i1_hbm, t0_hbm, t1_hbm, o_hbm,
             idx_v, rows_v, gsem, wsem0, wsem1) = refs
            work = ((i0_hbm, t0_hbm, o_hbm, 0),
                    (i1_hbm, t1_hbm, o_hbm, _EMB))
        else:
            (i2_hbm, t2_hbm, o_hbm,
             idx_v, rows_v, gsem, wsem0, wsem1) = refs
            work = ((i2_hbm, t2_hbm, o_hbm, 0),)
        wid = lax.axis_index("s") * _NC + lax.axis_index("c")
        gbase = start + wid * rows_per_w
        obase = wid * rows_per_w

        for ih, th, oh, col in work:
            @pl.loop(0, nch // 2)
            def _(k):
                for p, wsem in ((0, wsem0), (1, wsem1)):
                    c = 2 * k + p
                    r0 = gbase + c * ch
                    q0 = obase + c * ch
                    pltpu.sync_copy(ih.at[pl.ds(r0, ch)], idx_v.at[p])

                    @pl.when(k > 0)
                    def _():
                        pltpu.make_async_copy(
                            rows_v.at[p],
                            oh.at[pl.ds(q0, ch), pl.ds(col, _EMB)],
                            wsem).wait()

                    cps = []
                    for j in range(ch_rows):
                        cps.append(pltpu.async_copy(
                            th.at[idx_v.at[p, pl.ds(j * _IDXROW, _IDXROW)]],
                            rows_v.at[p, pl.ds(j * _IDXROW, _IDXROW)], gsem))
                    for cp in cps:
                        cp.wait()
                    pltpu.async_copy(
                        rows_v.at[p],
                        oh.at[pl.ds(q0, ch), pl.ds(col, _EMB)], wsem)

            for p, wsem in ((0, wsem0), (1, wsem1)):
                q0 = obase + (nch - 2 + p) * ch
                pltpu.make_async_copy(
                    rows_v.at[p],
                    oh.at[pl.ds(q0, ch), pl.ds(col, _EMB)], wsem).wait()
    return body


def _sc_gather(start, nrows, ch_rows, idxs, tabs):
    mesh = plsc.VectorSubcoreMesh(core_axis_name="c", subcore_axis_name="s",
                                  num_cores=_NC, num_subcores=_NS)
    ch = ch_rows * _IDXROW
    wide_ty = jax.ShapeDtypeStruct((nrows, 2 * _EMB), jnp.float32)
    k = pl.kernel(
        _make_sc_body(start, nrows, ch_rows, len(tabs) == 2),
        out_type=wide_ty,
        mesh=mesh,
        scratch_types=[
            pltpu.VMEM((2, ch), jnp.int32),
            pltpu.VMEM((2, ch, _EMB), jnp.float32),
            pltpu.SemaphoreType.DMA,
            pltpu.SemaphoreType.DMA,
            pltpu.SemaphoreType.DMA,
        ],
        compiler_params=pltpu.CompilerParams(use_tc_tiling_on_sc=False),
    )
    return k(*idxs, *tabs)


_TB = 8


def _make_fuse_body(t_base, aliased):
    def body(*refs):
        if aliased:
            _, c01_ref, c2_ref, nf0_ref, nf1_ref, seq_ref, wb_ref, \
                scal_ref, out_ref = refs
        else:
            c01_ref, c2_ref, nf0_ref, nf1_ref, seq_ref, wb_ref, \
                scal_ref, out_ref = refs
        i = pl.program_id(0)
        seq = seq_ref[...]
        iota_t = lax.broadcasted_iota(jnp.int32, (_T, _B), 0)
        mfull = (iota_t < seq).astype(jnp.float32)
        cnt = jnp.maximum(jnp.sum(mfull), 1.0)

        t0 = i * _TB + t_base
        iota_b = lax.broadcasted_iota(jnp.int32, (_TB, _B), 0) + t0
        mask = iota_b < seq

        c01 = jnp.swapaxes(c01_ref[...], 1, 2)
        c2 = jnp.swapaxes(c2_ref[...], 1, 2)[:, :_EMB, :]
        pieces = [c01, c2]
        for f, ref in enumerate((nf0_ref, nf1_ref)):
            xf = ref[...]
            s1 = jnp.sum(xf * mfull)
            s2 = jnp.sum(xf * xf * mfull)
            mean = s1 / cnt
            var = jnp.maximum(s2 / cnt - mean * mean, 0.0)
            rstd = lax.rsqrt(var + _EPS)
            gamma = scal_ref[2 * f]
            beta = scal_ref[2 * f + 1]
            x = ref[pl.ds(t0, _TB), :]
            xn = (x - mean) * (rstd * gamma) + beta
            y = jnp.where(mask, xn, x)
            w = wb_ref[2 * f, :]
            b = wb_ref[2 * f + 1, :]
            pieces.append(y[:, None, :] * w[None, :, None]
                          + b[None, :, None])
        out_ref[...] = jnp.concatenate(pieces, axis=1)
    return body


def _tc_fuse(t_base, nsteps, c01, c2, nf0, nf1, seq2d, wb, scal, prev=None):
    blk_base = t_base // _TB
    cat_spec = pl.BlockSpec((_TB, _B, 2 * _EMB), lambda i: (i, 0, 0))
    in_specs = [
        cat_spec, cat_spec,
        pl.BlockSpec((_T, _B), lambda i: (0, 0)),
        pl.BlockSpec((_T, _B), lambda i: (0, 0)),
        pl.BlockSpec((1, _B), lambda i: (0, 0)),
        pl.BlockSpec((4, 16), lambda i: (0, 0)),
        pl.BlockSpec(memory_space=pltpu.SMEM),
    ]
    args = [c01, c2, nf0, nf1, seq2d, wb, scal]
    aliases = {}
    if prev is not None:
        in_specs = [pl.BlockSpec(memory_space=pl.ANY)] + in_specs
        args = [prev] + args
        aliases = {0: 0}
    return pl.pallas_call(
        _make_fuse_body(t_base, prev is not None),
        grid=(nsteps,),
        in_specs=in_specs,
        out_specs=pl.BlockSpec((_TB, _OUT_D, _B),
                               lambda i: (i + blk_base, 0, 0)),
        out_shape=jax.ShapeDtypeStruct((_T, _OUT_D, _B), jnp.float32),
        input_output_aliases=aliases,
    )(*args)


def kernel(emb_feat_0, emb_feat_1, emb_feat_2, num_feat_0, num_feat_1,
           event_time, seq_lens, emb_table_0, emb_table_1, emb_table_2,
           bn_gamma_0, bn_beta_0, bn_gamma_1, bn_beta_1,
           lin_w_0, lin_b_0, lin_w_1, lin_b_1):
    idx0 = jnp.transpose(emb_feat_0.astype(jnp.int32), (1, 0)).reshape(_N)
    idx1 = jnp.transpose(emb_feat_1.astype(jnp.int32), (1, 0)).reshape(_N)
    idx2 = jnp.transpose(emb_feat_2.astype(jnp.int32), (1, 0)).reshape(_N)

    c01 = _sc_gather(0, _N, 5, (idx0, idx1), (emb_table_0, emb_table_1))
    c2 = _sc_gather(0, _N, 5, (idx2,), (emb_table_2,))

    seq2d = seq_lens.astype(jnp.int32).reshape(1, _B)
    wb = jnp.stack([lin_w_0[0].astype(jnp.float32),
                    lin_b_0.astype(jnp.float32),
                    lin_w_1[0].astype(jnp.float32),
                    lin_b_1.astype(jnp.float32)], axis=0)
    scal = jnp.stack([bn_gamma_0.astype(jnp.float32),
                      bn_beta_0.astype(jnp.float32),
                      bn_gamma_1.astype(jnp.float32),
                      bn_beta_1.astype(jnp.float32)])

    nf0_t = jnp.transpose(num_feat_0.astype(jnp.float32), (1, 0))
    nf1_t = jnp.transpose(num_feat_1.astype(jnp.float32), (1, 0))

    out_t = _tc_fuse(0, _T // _TB, c01.reshape(_T, _B, 2 * _EMB),
                     c2.reshape(_T, _B, 2 * _EMB),
                     nf0_t, nf1_t, seq2d, wb, scal)
    out = jnp.transpose(out_t, (2, 0, 1))
    return out, event_time.astype(jnp.float32)


# speedup vs baseline: 1.1664x; 1.0070x over previous
import jax
import jax.numpy as jnp
from jax import lax
from jax.experimental import pallas as pl
from jax.experimental.pallas import tpu as pltpu
from jax.experimental.pallas import tpu_sc as plsc

_B, _T = 1024, 200
_N = _B * _T
_VOCAB = 100000
_EMB = 64
_OUT_D = 3 * _EMB + 32
_EPS = 1e-5

_T0, _T1 = 80, 120
_N0, _N1 = _T0 * _B, _T1 * _B

_NC, _NS = 2, 16
_NW = _NC * _NS
_IDXROW = 128


def _make_sc_body(start, nrows, ch_rows, two_tables):
    rows_per_w = nrows // _NW
    ch = ch_rows * _IDXROW
    nch = rows_per_w // ch
    assert nch % 2 == 0

    def body(*refs):
        if two_tables:
            (i0_hbm, i1_hbm, t0_hbm, t1_hbm, o_hbm,
             idx_v, rows_v, gsem, wsem0, wsem1) = refs
            work = ((i0_hbm, t0_hbm, o_hbm, 0),
                    (i1_hbm, t1_hbm, o_hbm, _EMB))
        else:
            (i2_hbm, t2_hbm, o_hbm,
             idx_v, rows_v, gsem, wsem0, wsem1) = refs
            work = ((i2_hbm, t2_hbm, o_hbm, 0),)
        wid = lax.axis_index("s") * _NC + lax.axis_index("c")
        gbase = start + wid * rows_per_w
        obase = wid * rows_per_w

        for ih, th, oh, col in work:
            pltpu.sync_copy(ih.at[pl.ds(gbase, rows_per_w)], idx_v)

            @pl.loop(0, nch // 2)
            def _(k):
                for p, wsem in ((0, wsem0), (1, wsem1)):
                    c = 2 * k + p
                    q0 = obase + c * ch

                    @pl.when(k > 0)
                    def _():
                        pltpu.make_async_copy(
                            rows_v.at[p],
                            oh.at[pl.ds(q0, ch), pl.ds(col, _EMB)],
                            wsem).wait()

                    cps = []
                    for j in range(ch_rows):
                        off = c * ch + j * _IDXROW
                        cps.append(pltpu.async_copy(
                            th.at[idx_v.at[pl.ds(off, _IDXROW)]],
                            rows_v.at[p, pl.ds(j * _IDXROW, _IDXROW)], gsem))
                    for cp in cps:
                        cp.wait()
                    pltpu.async_copy(
                        rows_v.at[p],
                        oh.at[pl.ds(q0, ch), pl.ds(col, _EMB)], wsem)

            for p, wsem in ((0, wsem0), (1, wsem1)):
                q0 = obase + (nch - 2 + p) * ch
                pltpu.make_async_copy(
                    rows_v.at[p],
                    oh.at[pl.ds(q0, ch), pl.ds(col, _EMB)], wsem).wait()
    return body


def _sc_gather(start, nrows, ch_rows, idxs, tabs):
    mesh = plsc.VectorSubcoreMesh(core_axis_name="c", subcore_axis_name="s",
                                  num_cores=_NC, num_subcores=_NS)
    ch = ch_rows * _IDXROW
    wide_ty = jax.ShapeDtypeStruct((nrows, 2 * _EMB), jnp.float32)
    k = pl.kernel(
        _make_sc_body(start, nrows, ch_rows, len(tabs) == 2),
        out_type=wide_ty,
        mesh=mesh,
        scratch_types=[
            pltpu.VMEM((nrows // _NW,), jnp.int32),
            pltpu.VMEM((2, ch, _EMB), jnp.float32),
            pltpu.SemaphoreType.DMA,
            pltpu.SemaphoreType.DMA,
            pltpu.SemaphoreType.DMA,
        ],
        compiler_params=pltpu.CompilerParams(use_tc_tiling_on_sc=False),
    )
    return k(*idxs, *tabs)


_TB = 8


def _make_fuse_body(t_base, aliased):
    def body(*refs):
        if aliased:
            _, c01_ref, c2_ref, nf0_ref, nf1_ref, seq_ref, wb_ref, \
                scal_ref, out_ref = refs
        else:
            c01_ref, c2_ref, nf0_ref, nf1_ref, seq_ref, wb_ref, \
                scal_ref, out_ref = refs
        i = pl.program_id(0)
        seq = seq_ref[...]
        iota_t = lax.broadcasted_iota(jnp.int32, (_T, _B), 0)
        mfull = (iota_t < seq).astype(jnp.float32)
        cnt = jnp.maximum(jnp.sum(mfull), 1.0)

        t0 = i * _TB + t_base
        iota_b = lax.broadcasted_iota(jnp.int32, (_TB, _B), 0) + t0
        mask = iota_b < seq

        c01 = jnp.swapaxes(c01_ref[...], 1, 2)
        c2 = jnp.swapaxes(c2_ref[...], 1, 2)[:, :_EMB, :]
        pieces = [c01, c2]
        for f, ref in enumerate((nf0_ref, nf1_ref)):
            xf = ref[...]
            s1 = jnp.sum(xf * mfull)
            s2 = jnp.sum(xf * xf * mfull)
            mean = s1 / cnt
            var = jnp.maximum(s2 / cnt - mean * mean, 0.0)
            rstd = lax.rsqrt(var + _EPS)
            gamma = scal_ref[2 * f]
            beta = scal_ref[2 * f + 1]
            x = ref[pl.ds(t0, _TB), :]
            xn = (x - mean) * (rstd * gamma) + beta
            y = jnp.where(mask, xn, x)
            w = wb_ref[2 * f, :]
            b = wb_ref[2 * f + 1, :]
            pieces.append(y[:, None, :] * w[None, :, None]
                          + b[None, :, None])
        out_ref[...] = jnp.concatenate(pieces, axis=1)
    return body


def _tc_fuse(t_base, nsteps, c01, c2, nf0, nf1, seq2d, wb, scal, prev=None):
    blk_base = t_base // _TB
    cat_spec = pl.BlockSpec((_TB, _B, 2 * _EMB), lambda i: (i, 0, 0))
    in_specs = [
        cat_spec, cat_spec,
        pl.BlockSpec((_T, _B), lambda i: (0, 0)),
        pl.BlockSpec((_T, _B), lambda i: (0, 0)),
        pl.BlockSpec((1, _B), lambda i: (0, 0)),
        pl.BlockSpec((4, 16), lambda i: (0, 0)),
        pl.BlockSpec(memory_space=pltpu.SMEM),
    ]
    args = [c01, c2, nf0, nf1, seq2d, wb, scal]
    aliases = {}
    if prev is not None:
        in_specs = [pl.BlockSpec(memory_space=pl.ANY)] + in_specs
        args = [prev] + args
        aliases = {0: 0}
    return pl.pallas_call(
        _make_fuse_body(t_base, prev is not None),
        grid=(nsteps,),
        in_specs=in_specs,
        out_specs=pl.BlockSpec((_TB, _OUT_D, _B),
                               lambda i: (i + blk_base, 0, 0)),
        out_shape=jax.ShapeDtypeStruct((_T, _OUT_D, _B), jnp.float32),
        input_output_aliases=aliases,
    )(*args)


def kernel(emb_feat_0, emb_feat_1, emb_feat_2, num_feat_0, num_feat_1,
           event_time, seq_lens, emb_table_0, emb_table_1, emb_table_2,
           bn_gamma_0, bn_beta_0, bn_gamma_1, bn_beta_1,
           lin_w_0, lin_b_0, lin_w_1, lin_b_1):
    idx0 = jnp.transpose(emb_feat_0.astype(jnp.int32), (1, 0)).reshape(_N)
    idx1 = jnp.transpose(emb_feat_1.astype(jnp.int32), (1, 0)).reshape(_N)
    idx2 = jnp.transpose(emb_feat_2.astype(jnp.int32), (1, 0)).reshape(_N)

    c01 = _sc_gather(0, _N, 5, (idx0, idx1), (emb_table_0, emb_table_1))
    c2 = _sc_gather(0, _N, 5, (idx2,), (emb_table_2,))

    seq2d = seq_lens.astype(jnp.int32).reshape(1, _B)
    wb = jnp.stack([lin_w_0[0].astype(jnp.float32),
                    lin_b_0.astype(jnp.float32),
                    lin_w_1[0].astype(jnp.float32),
                    lin_b_1.astype(jnp.float32)], axis=0)
    scal = jnp.stack([bn_gamma_0.astype(jnp.float32),
                      bn_beta_0.astype(jnp.float32),
                      bn_gamma_1.astype(jnp.float32),
                      bn_beta_1.astype(jnp.float32)])

    nf0_t = jnp.transpose(num_feat_0.astype(jnp.float32), (1, 0))
    nf1_t = jnp.transpose(num_feat_1.astype(jnp.float32), (1, 0))

    out_t = _tc_fuse(0, _T // _TB, c01.reshape(_T, _B, 2 * _EMB),
                     c2.reshape(_T, _B, 2 * _EMB),
                     nf0_t, nf1_t, seq2d, wb, scal)
    out = jnp.transpose(out_t, (2, 0, 1))
    return out, event_time.astype(jnp.float32)
